# Initial kernel scaffold; baseline (speedup 1.0000x reference)
import jax
import jax.numpy as jnp
from jax.experimental import pallas as pl


def kernel(x, shared_up, shared_down, routed_up, routed_down, router_w):
    raise NotImplementedError("write your pallas kernel here")


# speedup vs baseline: 3.0701x; 3.0701x over previous
import functools

import jax
import jax.numpy as jnp
from jax import lax
from jax.experimental import pallas as pl
from jax.experimental.pallas import tpu as pltpu
from jax.experimental.pallas import tpu_sc as plsc

H, E, NS, NR, TOPK, T = 768, 3072, 2, 8, 2, 2048
BM = 512
NT = 16
S = NT * BM
EC = 1536
NSC = 16
TPW = T // NSC
NW = 32
SPW = S // NW
LPS = S // NSC


def _gelu_tc(v):
    return 0.5 * v * (1.0 + lax.erf(v * 0.7071067811865476))


def _shared_router(x, shared_up, shared_down, router_w):
    nec = E // EC

    def body(x_ref, up_ref, dn_ref, rw_ref, base_ref, lt_ref):
        n = pl.program_id(1)
        xb = x_ref[...]
        h = lax.dot_general(xb, up_ref[0], (((1,), (1,)), ((), ())),
                            preferred_element_type=jnp.float32)
        h = _gelu_tc(h)
        contrib = lax.dot_general(h, dn_ref[0], (((1,), (1,)), ((), ())),
                                  preferred_element_type=jnp.float32)

        @pl.when(n == 0)
        def _():
            base_ref[...] = xb + contrib
            lt_ref[...] = lax.dot_general(rw_ref[...], xb,
                                          (((1,), (1,)), ((), ())),
                                          preferred_element_type=jnp.float32)

        @pl.when(n != 0)
        def _():
            base_ref[...] = base_ref[...] + contrib

    return pl.pallas_call(
        body,
        grid=(T // BM, NS * nec),
        in_specs=[
            pl.BlockSpec((BM, H), lambda m, n: (m, 0)),
            pl.BlockSpec((1, EC, H), lambda m, n: (n // nec, n % nec, 0)),
            pl.BlockSpec((1, H, EC), lambda m, n: (n // nec, 0, n % nec)),
            pl.BlockSpec((NR, H), lambda m, n: (0, 0)),
        ],
        out_specs=[
            pl.BlockSpec((BM, H), lambda m, n: (m, 0)),
            pl.BlockSpec((NR, BM), lambda m, n: (0, m)),
        ],
        out_shape=[
            jax.ShapeDtypeStruct((T, H), jnp.float32),
            jax.ShapeDtypeStruct((NR, T), jnp.float32),
        ],
    )(x, shared_up, shared_down, router_w)


def _routing(lt):
    mesh = plsc.VectorSubcoreMesh(core_axis_name="c", subcore_axis_name="s",
                                  num_cores=1, num_subcores=16)
    out_type = (
        jax.ShapeDtypeStruct((S,), jnp.int32),
        jax.ShapeDtypeStruct((S,), jnp.float32),
        jax.ShapeDtypeStruct((2 * T,), jnp.int32),
        jax.ShapeDtypeStruct((NT,), jnp.int32),
        jax.ShapeDtypeStruct((NT,), jnp.int32),
    )
    scratch = [
        pltpu.VMEM((NR, TPW), jnp.float32),
        pltpu.VMEM((2, TPW), jnp.int32),
        pltpu.VMEM((2, TPW), jnp.int32),
        pltpu.VMEM((2, TPW), jnp.float32),
        pltpu.VMEM((2, TPW), jnp.int32),
        pltpu.VMEM((2, TPW), jnp.int32),
        pltpu.VMEM((2 * TPW,), jnp.int32),
        pltpu.VMEM((16,), jnp.int32),
        pltpu.VMEM((16,), jnp.int32),
        pltpu.VMEM((NSC * 16,), jnp.int32),
        pltpu.VMEM((LPS,), jnp.int32),
        pltpu.VMEM((LPS,), jnp.float32),
        pltpu.VMEM((16,), jnp.int32),
        pltpu.VMEM((16,), jnp.int32),
        pltpu.VMEM_SHARED((S,), jnp.int32),
        pltpu.VMEM_SHARED((S,), jnp.float32),
        pltpu.VMEM_SHARED((NSC * 16,), jnp.int32),
    ]

    @functools.partial(pl.kernel, mesh=mesh, out_type=out_type,
                       compiler_params=pltpu.CompilerParams(
                           needs_layout_passes=False),
                       scratch_types=scratch)
    def body(lt_h, perm_h, gates_h, inv_h, eot_h, val_h,
             lg_v, ev_v, rv_v, pv_v, tv_v, idx_v, inv_v, histm_v, hidx_v,
             histall_v, tmpi_v, tmpf_v, eot_v, val_v, perm_s, gates_s,
             hist_s):
        w = lax.axis_index("s")
        lane = lax.iota(jnp.int32, 16)
        zi = jnp.zeros((16,), jnp.int32)
        ones = jnp.ones((16,), jnp.int32)
        ngrp = TPW // 16

        for jj in range(LPS // 16):
            tmpi_v[pl.ds(jj * 16, 16)] = zi
            tmpf_v[pl.ds(jj * 16, 16)] = jnp.zeros((16,), jnp.float32)
        stripe = w * LPS
        pltpu.sync_copy(tmpi_v, perm_s.at[pl.ds(stripe, LPS)])
        pltpu.sync_copy(tmpf_v, gates_s.at[pl.ds(stripe, LPS)])

        pltpu.sync_copy(lt_h.at[:, pl.ds(w * TPW, TPW)], lg_v)

        cnt = [zi for _ in range(NR)]
        for g in range(ngrp):
            sl = pl.ds(g * 16, 16)
            l = [lg_v[e, sl] for e in range(NR)]
            m = l[0]
            for e in range(1, NR):
                m = jnp.maximum(m, l[e])
            p = [jnp.exp(l[e] - m) for e in range(NR)]
            ssum = p[0]
            for e in range(1, NR):
                ssum = ssum + p[e]
            v1 = p[0]
            e1 = zi
            for e in range(1, NR):
                b = p[e] > v1
                e1 = jnp.where(b, e, e1)
                v1 = jnp.where(b, p[e], v1)
            v2 = jnp.where(e1 == 0, -1.0, p[0])
            e2 = zi
            for e in range(1, NR):
                pe = jnp.where(e1 == e, -1.0, p[e])
                b = pe > v2
                e2 = jnp.where(b, e, e2)
                v2 = jnp.where(b, pe, v2)
            pv_v[0, sl] = v1 / ssum
            pv_v[1, sl] = v2 / ssum
            toks = w * TPW + g * 16 + lane
            tv_v[0, sl] = toks
            tv_v[1, sl] = toks
            ev_v[0, sl] = e1
            ev_v[1, sl] = e2
            for j, ev in ((0, e1), (1, e2)):
                r = zi
                for e in range(NR):
                    msk = ev == e
                    cum = plsc.cumsum(ones, mask=msk)
                    r = jnp.where(msk, cnt[e] + cum - 1, r)
                    cnt[e] = cnt[e] + plsc.all_reduce_population_count(msk)
                rv_v[j, sl] = r

        hist = zi
        for e in range(NR):
            hist = jnp.where(lane == e, cnt[e], hist)
        histm_v[...] = hist
        hidx_v[...] = w * 16 + lane
        pltpu.sync_copy(histm_v, hist_s.at[hidx_v])
        plsc.subcore_barrier()
        pltpu.sync_copy(hist_s, histall_v)

        pre = zi
        tot = zi
        for wp in range(NSC):
            hrow = histall_v[pl.ds(wp * 16, 16)]
            pre = pre + jnp.where(wp < w, hrow, zi)
            tot = tot + hrow
        blocks = jnp.right_shift(tot + (BM - 1), 9)
        incl = plsc.cumsum(blocks)
        excl = incl - blocks
        base_vec = excl * BM + pre
        bes = [jnp.sum(jnp.where(lane == e, base_vec, zi)) for e in range(NR)]

        for j in range(2):
            for g in range(ngrp):
                sl = pl.ds(g * 16, 16)
                ev = ev_v[j, sl]
                r = rv_v[j, sl]
                bsel = zi
                for e in range(NR):
                    bsel = jnp.where(ev == e, bes[e], bsel)
                pos = bsel + r
                idx_v[j, sl] = pos
                plsc.store_scatter(inv_v, [2 * (g * 16 + lane) + j], pos)
        for j in range(2):
            pltpu.sync_copy(tv_v.at[j], perm_s.at[idx_v.at[j]])
            pltpu.sync_copy(pv_v.at[j], gates_s.at[idx_v.at[j]])
        plsc.subcore_barrier()

        pltpu.sync_copy(perm_s.at[pl.ds(stripe, LPS)], tmpi_v)
        pltpu.sync_copy(tmpi_v, perm_h.at[pl.ds(stripe, LPS)])
        pltpu.sync_copy(gates_s.at[pl.ds(stripe, LPS)], tmpf_v)
        pltpu.sync_copy(tmpf_v, gates_h.at[pl.ds(stripe, LPS)])
        pltpu.sync_copy(inv_v, inv_h.at[pl.ds(2 * w * TPW, 2 * TPW)])

        @pl.when(w == 0)
        def _():
            laste = jnp.max(jnp.where(tot > 0, lane, zi))
            eo = zi
            va = zi
            for e in range(NR):
                b_e = jnp.sum(jnp.where(lane == e, blocks, zi))
                s_e = jnp.sum(jnp.where(lane == e, excl, zi))
                inr = (lane >= s_e) & (lane < s_e + b_e)
                eo = jnp.where(inr, e, eo)
                va = jnp.where(inr, 1, va)
            eo = jnp.where(va == 1, eo, laste)
            eot_v[...] = eo
            val_v[...] = va
            pltpu.sync_copy(eot_v, eot_h)
            pltpu.sync_copy(val_v, val_h)

    return body(lt)


def _gather(x, perm, valid):
    mesh = plsc.VectorSubcoreMesh(core_axis_name="c", subcore_axis_name="s",
                                  num_cores=2, num_subcores=16)

    @functools.partial(
        pl.kernel, mesh=mesh,
        out_type=jax.ShapeDtypeStruct((S, H), jnp.float32),
        compiler_params=pltpu.CompilerParams(needs_layout_passes=False),
        scratch_types=[
            pltpu.VMEM((128,), jnp.int32),
            pltpu.VMEM((128, H), jnp.float32),
            pltpu.VMEM((16,), jnp.int32),
        ])
    def body(x_h, perm_h, val_h, xs_h, idx_v, rows_v, val_v):
        w = lax.axis_index("s") * 2 + lax.axis_index("c")
        pltpu.sync_copy(val_h, val_v)
        used = jnp.sum(val_v[...])
        for ch in range(SPW // 128):
            base = w * SPW + ch * 128

            @pl.when(base < used * BM)
            def _():
                pltpu.sync_copy(perm_h.at[pl.ds(base, 128)], idx_v)
                pltpu.sync_copy(x_h.at[idx_v], rows_v)
                pltpu.sync_copy(rows_v, xs_h.at[pl.ds(base, 128)])

    return body(x, perm, valid)


def _routed(xs, routed_up, routed_down, gates3, eot, valid):
    def body(eot_ref, val_ref, xs_ref, up_ref, dn_ref, g_ref, out_ref):
        i = pl.program_id(0)

        @pl.when(val_ref[i] == 1)
        def _():
            h = lax.dot_general(xs_ref[...], up_ref[0], (((1,), (1,)), ((), ())),
                                preferred_element_type=jnp.float32)
            h = _gelu_tc(h)
            y = lax.dot_general(h, dn_ref[0], (((1,), (1,)), ((), ())),
                                preferred_element_type=jnp.float32)
            out_ref[...] = y * g_ref[0, 0, :][:, None]

        @pl.when(val_ref[i] == 0)
        def _():
            out_ref[...] = jnp.zeros_like(out_ref)

    grid_spec = pltpu.PrefetchScalarGridSpec(
        num_scalar_prefetch=2,
        grid=(NT,),
        in_specs=[
            pl.BlockSpec((BM, H), lambda i, eot, val: (i, 0)),
            pl.BlockSpec((1, E, H), lambda i, eot, val: (eot[i], 0, 0)),
            pl.BlockSpec((1, H, E), lambda i, eot, val: (eot[i], 0, 0)),
            pl.BlockSpec((1, 1, BM), lambda i, eot, val: (i, 0, 0)),
        ],
        out_specs=pl.BlockSpec((BM, H), lambda i, eot, val: (i, 0)),
    )
    return pl.pallas_call(
        body, grid_spec=grid_spec,
        out_shape=jax.ShapeDtypeStruct((S, H), jnp.float32),
    )(eot, valid, xs, routed_up, routed_down, gates3)


def _combine(base_act, ysg, inv):
    mesh = plsc.VectorSubcoreMesh(core_axis_name="c", subcore_axis_name="s",
                                  num_cores=2, num_subcores=16)
    tpw = T // NW

    @functools.partial(
        pl.kernel, mesh=mesh,
        out_type=jax.ShapeDtypeStruct((T, H), jnp.float32),
        compiler_params=pltpu.CompilerParams(needs_layout_passes=False),
        scratch_types=[
            pltpu.VMEM((64,), jnp.int32),
            pltpu.VMEM((64, H), jnp.float32),
            pltpu.VMEM((32, H), jnp.float32),
        ])
    def body(b_h, y_h, inv_h, out_h, inv_v, rows_v, acc_v):
        w = lax.axis_index("s") * 2 + lax.axis_index("c")
        for ch in range(tpw // 32):
            tb = w * tpw + ch * 32
            pltpu.sync_copy(inv_h.at[pl.ds(2 * tb, 64)], inv_v)
            pltpu.sync_copy(y_h.at[inv_v], rows_v)
            pltpu.sync_copy(b_h.at[pl.ds(tb, 32)], acc_v)

            def tok(t, carry):
                for jj in range(H // 16):
                    sl = pl.ds(jj * 16, 16)
                    acc_v[t, sl] = acc_v[t, sl] + rows_v[2 * t, sl] + rows_v[2 * t + 1, sl]
                return carry

            lax.fori_loop(0, 32, tok, 0)
            pltpu.sync_copy(acc_v, out_h.at[pl.ds(tb, 32)])

    return body(base_act, ysg, inv)


def kernel(x, shared_up, shared_down, routed_up, routed_down, router_w):
    base_act, lt = _shared_router(x, shared_up, shared_down, router_w)
    perm, gates, inv, eot, valid = _routing(lt)
    xs = _gather(x, perm, valid)
    gates3 = gates.reshape(NT, 1, BM)
    ysg = _routed(xs, routed_up, routed_down, gates3, eot, valid)
    return _combine(base_act, ysg, inv)
